# baseline (speedup 1.0000x reference)
import jax
import jax.numpy as jnp
from jax.experimental import pallas as pl
from jax.experimental.pallas import tpu as pltpu

_H = 32
_W = 32
_HW = _H * _W
_CP = 640
_CS = 160
_PAD = _W + 1
_ROWS = _HW + 2 * _PAD
_HM = 2
_EMB_OFF = _HM + 4
_EMB = 512


def _bn_lrelu(acc, s_ref, b_ref):
    y = acc * s_ref[0] + b_ref[0]
    return jnp.where(y > 0.0, y, 0.01 * y)


def _fused_kernel(cols_ref, ws_ref, ss_ref, bs_ref,
                  w1_ref, s1_ref, b1_ref,
                  w2_ref, s2_ref, b2_ref,
                  w3_ref, s3_ref, b3_ref,
                  o_ref, bufs, bufa, bufb):
    f32 = jnp.float32
    bf = jnp.bfloat16

    for buf in (bufs, bufa, bufb):
        buf[0:_PAD, :] = jnp.zeros((_PAD, buf.shape[1]), bf)
        buf[_PAD + _HW:_ROWS, :] = jnp.zeros((_PAD, buf.shape[1]), bf)

    x = jax.lax.broadcasted_iota(jnp.int32, (_HW, 1), 0) % _W
    m0 = x != 0
    m2 = x != (_W - 1)

    def conv3x3(buf, w_ref, kc):
        acc = None
        for t in range(9):
            dy, dx = t // 3, t % 3
            r0 = dy * _W + dx
            slab = buf[r0:r0 + _HW, :]
            if dx == 0:
                slab = jnp.where(m0, slab, jnp.zeros((), bf))
            elif dx == 2:
                slab = jnp.where(m2, slab, jnp.zeros((), bf))
            d = jnp.dot(slab, w_ref[t * kc:(t + 1) * kc, :],
                        preferred_element_type=f32)
            acc = d if acc is None else acc + d
        return acc

    y = jnp.dot(cols_ref[0], ws_ref[...], preferred_element_type=f32)
    y = _bn_lrelu(y, ss_ref, bs_ref)
    bufs[_PAD:_PAD + _HW, :] = y.astype(bf)

    y = _bn_lrelu(conv3x3(bufs, w1_ref, _CS), s1_ref, b1_ref)
    bufa[_PAD:_PAD + _HW, :] = y.astype(bf)
    y = _bn_lrelu(conv3x3(bufa, w2_ref, _CP), s2_ref, b2_ref)
    bufb[_PAD:_PAD + _HW, :] = y.astype(bf)
    y = _bn_lrelu(conv3x3(bufb, w3_ref, _CP), s3_ref, b3_ref)

    ch = jax.lax.broadcasted_iota(jnp.int32, (_HW, _CP), 1)
    is_hm = ch < _HM
    is_emb = (ch >= _EMB_OFF) & (ch < _EMB_OFF + _EMB)

    zhm = jnp.where(is_hm, y, -jnp.inf)
    m = jnp.max(zhm, axis=-1, keepdims=True)
    e = jnp.exp(zhm - m)
    hm = e / jnp.sum(e, axis=-1, keepdims=True)

    sq = jnp.where(is_emb, y * y, 0.0)
    nrm = jnp.maximum(jnp.sqrt(jnp.sum(sq, axis=-1, keepdims=True)), 1e-12)
    emb = y / nrm

    out = jnp.where(is_hm, hm, jnp.where(is_emb, emb, y))
    o_ref[...] = out.reshape(1, _HW, _CP)


def _stem_cols(x_nhwc):
    N, H, W, Cimg = x_nhwc.shape
    xp = jnp.pad(x_nhwc, ((0, 0), (1, 1), (1, 1), (0, 0)))
    taps = [xp[:, dy:dy + H, dx:dx + W, :]
            for dy in range(3) for dx in range(3)]
    cols = jnp.concatenate(taps, axis=-1).reshape(N, H * W, 9 * Cimg)
    kpad = (-(9 * Cimg)) % 8
    if kpad:
        cols = jnp.pad(cols, ((0, 0), (0, 0), (0, kpad)))
    return cols.astype(jnp.bfloat16)


def kernel(img, ws, ss, bs, w1, s1, b1, w2, s2, b2, w3, s3, b3):
    N = img.shape[0]
    x = jnp.transpose(img, (0, 2, 3, 1)).astype(jnp.float32)
    cols = _stem_cols(x)
    KS = cols.shape[-1]

    ws_n = ws[:, :_CS]
    ss_n = ss[:, :_CS]
    bs_n = bs[:, :_CS]
    w1_n = w1.reshape(9, _CP, _CP)[:, :_CS, :].reshape(9 * _CS, _CP)

    wspec = lambda shape: pl.BlockSpec(shape, lambda n: (0, 0))
    out = pl.pallas_call(
        _fused_kernel,
        out_shape=jax.ShapeDtypeStruct((N, _HW, _CP), jnp.float32),
        grid=(N,),
        in_specs=[
            pl.BlockSpec((1, _HW, KS), lambda n: (n, 0, 0)),
            wspec((KS, _CS)), wspec((1, _CS)), wspec((1, _CS)),
            wspec((9 * _CS, _CP)), wspec((1, _CP)), wspec((1, _CP)),
            wspec((9 * _CP, _CP)), wspec((1, _CP)), wspec((1, _CP)),
            wspec((9 * _CP, _CP)), wspec((1, _CP)), wspec((1, _CP)),
        ],
        out_specs=pl.BlockSpec((1, _HW, _CP), lambda n: (n, 0, 0)),
        scratch_shapes=[
            pltpu.VMEM((_ROWS, _CS), jnp.bfloat16),
            pltpu.VMEM((_ROWS, _CP), jnp.bfloat16),
            pltpu.VMEM((_ROWS, _CP), jnp.bfloat16),
        ],
        compiler_params=pltpu.CompilerParams(
            dimension_semantics=("parallel",)),
    )(cols, ws_n, ss_n, bs_n,
      w1_n, s1, b1, w2, s2, b2, w3, s3, b3)

    out = out.reshape(N, _H, _W, _CP)
    heatmap = out[..., 0:_HM]
    offset = out[..., _HM:_HM + 2]
    wh = out[..., _HM + 2:_HM + 4]
    emb = out[..., _EMB_OFF:_EMB_OFF + _EMB]
    to_nchw = lambda t: jnp.transpose(t, (0, 3, 1, 2))
    return to_nchw(heatmap), to_nchw(offset), to_nchw(wh), to_nchw(emb)


# speedup vs baseline: 1.1896x; 1.1896x over previous
import jax
import jax.numpy as jnp
from jax.experimental import pallas as pl
from jax.experimental.pallas import tpu as pltpu

_H = 32
_W = 32
_HW = _H * _W
_CP = 640
_CS = 160
_PAD = _W + 1
_ROWS = _HW + 2 * _PAD
_HM = 2
_EMB = 512
_NS = 128


def _bn_lrelu(acc, s_ref, b_ref):
    y = acc * s_ref[0] + b_ref[0]
    return jnp.where(y > 0.0, y, 0.01 * y)


def _fused_kernel(cols_ref, ws_ref, ss_ref, bs_ref,
                  w1_ref, s1_ref, b1_ref,
                  w2_ref, s2_ref, b2_ref,
                  we_ref, se_ref, be_ref,
                  wh_ref, sh_ref, bh_ref,
                  oe_ref, oh_ref,
                  s0, s1b, s2b, a0, a1, a2, c0, c1, c2, n0, n1, n2):
    f32 = jnp.float32
    bf = jnp.bfloat16

    for b0, b1, b2 in ((s0, s1b, s2b), (a0, a1, a2), (c0, c1, c2),
                       (n0, n1, n2)):
        for buf in (b0, b1, b2):
            w = buf.shape[1]
            buf[0:_PAD, :] = jnp.zeros((_PAD, w), bf)
            buf[_ROWS - _PAD - 2:_ROWS, :] = jnp.zeros((_PAD + 2, w), bf)

    col = jax.lax.broadcasted_iota(jnp.int32, (_HW, 1), 0) % _W
    not_last = col != (_W - 1)
    not_first = col != 0
    zero = jnp.zeros((), bf)

    def store3(y, b0, b1, b2):
        yb = y.astype(bf)
        b0[_PAD:_PAD + _HW, :] = jnp.where(not_last, yb, zero)
        b1[_PAD - 1:_PAD - 1 + _HW, :] = yb
        b2[_PAD - 2:_PAD - 2 + _HW, :] = jnp.where(not_first, yb, zero)

    def conv3x3(b0, b1, b2, w_ref, kc):
        bufs = (b0, b1, b2)
        acc = None
        for t in range(9):
            dy, dx = t // 3, t % 3
            slab = bufs[dx][dy * _W:dy * _W + _HW, :]
            d = jnp.dot(slab, w_ref[t * kc:(t + 1) * kc, :],
                        preferred_element_type=f32)
            acc = d if acc is None else acc + d
        return acc

    y = jnp.dot(cols_ref[0], ws_ref[...], preferred_element_type=f32)
    store3(_bn_lrelu(y, ss_ref, bs_ref), s0, s1b, s2b)

    y = _bn_lrelu(conv3x3(s0, s1b, s2b, w1_ref, _CS), s1_ref, b1_ref)
    store3(y, a0, a1, a2)

    y = _bn_lrelu(conv3x3(a0, a1, a2, w2_ref, _CP), s2_ref, b2_ref)
    store3(y, c0, c1, c2)
    store3(y[:, :_CS], n0, n1, n2)

    ye = _bn_lrelu(conv3x3(n0, n1, n2, we_ref, _CS), se_ref, be_ref)
    yh = _bn_lrelu(conv3x3(c0, c1, c2, wh_ref, _CP), sh_ref, bh_ref)

    ch = jax.lax.broadcasted_iota(jnp.int32, (_HW, _NS), 1)
    is_hm = ch < _HM
    zhm = jnp.where(is_hm, yh, -jnp.inf)
    m = jnp.max(zhm, axis=-1, keepdims=True)
    e = jnp.exp(zhm - m)
    hm = e / jnp.sum(e, axis=-1, keepdims=True)
    oh_ref[...] = jnp.where(is_hm, hm, yh).reshape(1, _HW, _NS)

    nrm = jnp.maximum(
        jnp.sqrt(jnp.sum(ye * ye, axis=-1, keepdims=True)), 1e-12)
    oe_ref[...] = (ye / nrm).reshape(1, _HW, _EMB)


def _stem_cols(x_nhwc):
    N, H, W, Cimg = x_nhwc.shape
    xp = jnp.pad(x_nhwc, ((0, 0), (1, 1), (1, 1), (0, 0)))
    taps = [xp[:, dy:dy + H, dx:dx + W, :]
            for dy in range(3) for dx in range(3)]
    cols = jnp.concatenate(taps, axis=-1).reshape(N, H * W, 9 * Cimg)
    kpad = (-(9 * Cimg)) % 8
    if kpad:
        cols = jnp.pad(cols, ((0, 0), (0, 0), (0, kpad)))
    return cols.astype(jnp.bfloat16)


def kernel(img, ws, ss, bs, w1, s1, b1, w2, s2, b2, w3, s3, b3):
    N = img.shape[0]
    x = jnp.transpose(img, (0, 2, 3, 1)).astype(jnp.float32)
    cols = _stem_cols(x)
    KS = cols.shape[-1]
    bf = jnp.bfloat16
    emb_off = _HM + 4

    ws_n = ws[:, :_CS]
    ss_n = ss[:, :_CS]
    bs_n = bs[:, :_CS]
    w1_n = w1.reshape(9, _CP, _CP)[:, :_CS, :].reshape(9 * _CS, _CP)

    perm = jnp.concatenate([jnp.arange(3 * _CS, _CP), jnp.arange(3 * _CS)])
    w2p = w2.reshape(9, _CP, _CP)[:, :, perm].reshape(9 * _CP, _CP)
    s2p = s2[:, perm]
    b2p = b2[:, perm]
    w3p = w3.reshape(9, _CP, _CP)[:, perm, :]

    w_e = w3p[:, :_CS, emb_off:emb_off + _EMB].reshape(9 * _CS, _EMB)
    s_e = s3[:, emb_off:emb_off + _EMB]
    b_e = b3[:, emb_off:emb_off + _EMB]
    w_h = jnp.zeros((9, _CP, _NS), bf).at[:, :, :6].set(
        w3p[:, :, :6].astype(bf)).reshape(9 * _CP, _NS)
    s_h = jnp.ones((1, _NS), s3.dtype).at[:, :6].set(s3[:, :6])
    b_h = jnp.zeros((1, _NS), b3.dtype).at[:, :6].set(b3[:, :6])

    wspec = lambda shape: pl.BlockSpec(shape, lambda n: (0, 0))
    vm = lambda w: pltpu.VMEM((_ROWS, w), bf)
    oe, oh = pl.pallas_call(
        _fused_kernel,
        out_shape=(jax.ShapeDtypeStruct((N, _HW, _EMB), jnp.float32),
                   jax.ShapeDtypeStruct((N, _HW, _NS), jnp.float32)),
        grid=(N,),
        in_specs=[
            pl.BlockSpec((1, _HW, KS), lambda n: (n, 0, 0)),
            wspec((KS, _CS)), wspec((1, _CS)), wspec((1, _CS)),
            wspec((9 * _CS, _CP)), wspec((1, _CP)), wspec((1, _CP)),
            wspec((9 * _CP, _CP)), wspec((1, _CP)), wspec((1, _CP)),
            wspec((9 * _CS, _EMB)), wspec((1, _EMB)), wspec((1, _EMB)),
            wspec((9 * _CP, _NS)), wspec((1, _NS)), wspec((1, _NS)),
        ],
        out_specs=(pl.BlockSpec((1, _HW, _EMB), lambda n: (n, 0, 0)),
                   pl.BlockSpec((1, _HW, _NS), lambda n: (n, 0, 0))),
        scratch_shapes=[
            vm(_CS), vm(_CS), vm(_CS),
            vm(_CP), vm(_CP), vm(_CP),
            vm(_CP), vm(_CP), vm(_CP),
            vm(_CS), vm(_CS), vm(_CS),
        ],
        compiler_params=pltpu.CompilerParams(
            dimension_semantics=("parallel",)),
    )(cols, ws_n, ss_n, bs_n,
      w1_n.astype(bf), s1, b1, w2p.astype(bf), s2p, b2p,
      w_e.astype(bf), s_e, b_e, w_h, s_h, b_h)

    oh = oh.reshape(N, _H, _W, _NS)
    heatmap = oh[..., 0:_HM]
    offset = oh[..., _HM:_HM + 2]
    wh = oh[..., _HM + 2:_HM + 4]
    emb = oe.reshape(N, _H, _W, _EMB)
    to_nchw = lambda t: jnp.transpose(t, (0, 3, 1, 2))
    return to_nchw(heatmap), to_nchw(offset), to_nchw(wh), to_nchw(emb)


# speedup vs baseline: 1.5669x; 1.3172x over previous
import jax
import jax.numpy as jnp
from jax.experimental import pallas as pl
from jax.experimental.pallas import tpu as pltpu

_H = 32
_W = 32
_HW = _H * _W
_CP = 640
_CS = 160
_CSP = 256
_PAD = _W + 1
_ROWS = _HW + 2 * _PAD
_HM = 2
_EMB = 512
_NS = 128
_KN = 9 * _CSP
_KW = 9 * _CP


def _bn_lrelu(acc, s_ref, b_ref):
    y = acc * s_ref[0] + b_ref[0]
    return jnp.where(y > 0.0, y, 0.01 * y)


def _fused_kernel(cols_ref, ws_ref, ss_ref, bs_ref,
                  w1_ref, s1_ref, b1_ref,
                  w2_ref, s2_ref, b2_ref,
                  we_ref, se_ref, be_ref,
                  wh_ref, sh_ref, bh_ref,
                  oe_ref, oh_ref,
                  s0, s1b, s2b, a0, a1, a2, c0, c1, c2, n0, n1, n2, col):
    f32 = jnp.float32
    bf = jnp.bfloat16

    for b0, b1, b2 in ((s0, s1b, s2b), (a0, a1, a2), (c0, c1, c2),
                       (n0, n1, n2)):
        for buf in (b0, b1, b2):
            w = buf.shape[1]
            buf[0:_PAD, :] = jnp.zeros((_PAD, w), bf)
            buf[_ROWS - _PAD - 2:_ROWS, :] = jnp.zeros((_PAD + 2, w), bf)

    cidx = jax.lax.broadcasted_iota(jnp.int32, (_HW, 1), 0) % _W
    not_last = cidx != (_W - 1)
    not_first = cidx != 0
    zero = jnp.zeros((), bf)

    def store3(y, b0, b1, b2, padto=None):
        yb = y.astype(bf)
        if padto is not None:
            yb = jnp.concatenate(
                [yb, jnp.zeros((_HW, padto - yb.shape[1]), bf)], axis=1)
        b0[_PAD:_PAD + _HW, :] = jnp.where(not_last, yb, zero)
        b1[_PAD - 1:_PAD - 1 + _HW, :] = yb
        b2[_PAD - 2:_PAD - 2 + _HW, :] = jnp.where(not_first, yb, zero)

    def build_col(b0, b1, b2, w):
        bufs = (b0, b1, b2)
        for t in range(9):
            dy, dx = t // 3, t % 3
            col[:, t * w:(t + 1) * w] = bufs[dx][dy * _W:dy * _W + _HW, :]

    y = jnp.dot(cols_ref[0], ws_ref[...], preferred_element_type=f32)
    store3(_bn_lrelu(y, ss_ref, bs_ref), s0, s1b, s2b, padto=_CSP)

    build_col(s0, s1b, s2b, _CSP)
    y = jnp.dot(col[:, :_KN], w1_ref[...], preferred_element_type=f32)
    store3(_bn_lrelu(y, s1_ref, b1_ref), a0, a1, a2)

    build_col(a0, a1, a2, _CP)
    y = jnp.dot(col[...], w2_ref[...], preferred_element_type=f32)
    y = _bn_lrelu(y, s2_ref, b2_ref)
    store3(y, c0, c1, c2)
    store3(y[:, :_CS], n0, n1, n2, padto=_CSP)

    build_col(n0, n1, n2, _CSP)
    ye = jnp.dot(col[:, :_KN], we_ref[...], preferred_element_type=f32)
    ye = _bn_lrelu(ye, se_ref, be_ref)

    cbufs = (c0, c1, c2)
    acc = None
    for t in range(9):
        dy, dx = t // 3, t % 3
        slab = cbufs[dx][dy * _W:dy * _W + _HW, :]
        d = jnp.dot(slab, wh_ref[t * _CP:(t + 1) * _CP, :],
                    preferred_element_type=f32)
        acc = d if acc is None else acc + d
    yh = _bn_lrelu(acc, sh_ref, bh_ref)

    ch = jax.lax.broadcasted_iota(jnp.int32, (_HW, _NS), 1)
    is_hm = ch < _HM
    zhm = jnp.where(is_hm, yh, -jnp.inf)
    m = jnp.max(zhm, axis=-1, keepdims=True)
    e = jnp.exp(zhm - m)
    hm = e / jnp.sum(e, axis=-1, keepdims=True)
    oh_ref[...] = jnp.where(is_hm, hm, yh).reshape(1, _HW, _NS)

    nrm = jnp.maximum(
        jnp.sqrt(jnp.sum(ye * ye, axis=-1, keepdims=True)), 1e-12)
    oe_ref[...] = (ye / nrm).reshape(1, _HW, _EMB)


def _stem_cols(x_nhwc):
    N, H, W, Cimg = x_nhwc.shape
    xp = jnp.pad(x_nhwc, ((0, 0), (1, 1), (1, 1), (0, 0)))
    taps = [xp[:, dy:dy + H, dx:dx + W, :]
            for dy in range(3) for dx in range(3)]
    cols = jnp.concatenate(taps, axis=-1).reshape(N, H * W, 9 * Cimg)
    kpad = (-(9 * Cimg)) % 8
    if kpad:
        cols = jnp.pad(cols, ((0, 0), (0, 0), (0, kpad)))
    return cols.astype(jnp.bfloat16)


def kernel(img, ws, ss, bs, w1, s1, b1, w2, s2, b2, w3, s3, b3):
    N = img.shape[0]
    x = jnp.transpose(img, (0, 2, 3, 1)).astype(jnp.float32)
    cols = _stem_cols(x)
    KS = cols.shape[-1]
    bf = jnp.bfloat16
    emb_off = _HM + 4

    ws_n = ws[:, :_CS]
    ss_n = ss[:, :_CS]
    bs_n = bs[:, :_CS]
    w1_n = jnp.zeros((9, _CSP, _CP), bf).at[:, :_CS, :].set(
        w1.reshape(9, _CP, _CP)[:, :_CS, :].astype(bf)).reshape(_KN, _CP)

    perm = jnp.concatenate([jnp.arange(3 * _CS, _CP), jnp.arange(3 * _CS)])
    w2p = w2.reshape(9, _CP, _CP)[:, :, perm].reshape(_KW, _CP)
    s2p = s2[:, perm]
    b2p = b2[:, perm]
    w3p = w3.reshape(9, _CP, _CP)[:, perm, :]

    w_e = jnp.zeros((9, _CSP, _EMB), bf).at[:, :_CS, :].set(
        w3p[:, :_CS, emb_off:emb_off + _EMB].astype(bf)).reshape(_KN, _EMB)
    s_e = s3[:, emb_off:emb_off + _EMB]
    b_e = b3[:, emb_off:emb_off + _EMB]
    w_h = jnp.zeros((9, _CP, _NS), bf).at[:, :, :6].set(
        w3p[:, :, :6].astype(bf)).reshape(_KW, _NS)
    s_h = jnp.ones((1, _NS), s3.dtype).at[:, :6].set(s3[:, :6])
    b_h = jnp.zeros((1, _NS), b3.dtype).at[:, :6].set(b3[:, :6])

    wspec = lambda shape: pl.BlockSpec(shape, lambda n: (0, 0))
    vm = lambda w: pltpu.VMEM((_ROWS, w), bf)
    oe, oh = pl.pallas_call(
        _fused_kernel,
        out_shape=(jax.ShapeDtypeStruct((N, _HW, _EMB), jnp.float32),
                   jax.ShapeDtypeStruct((N, _HW, _NS), jnp.float32)),
        grid=(N,),
        in_specs=[
            pl.BlockSpec((1, _HW, KS), lambda n: (n, 0, 0)),
            wspec((KS, _CS)), wspec((1, _CS)), wspec((1, _CS)),
            wspec((_KN, _CP)), wspec((1, _CP)), wspec((1, _CP)),
            wspec((_KW, _CP)), wspec((1, _CP)), wspec((1, _CP)),
            wspec((_KN, _EMB)), wspec((1, _EMB)), wspec((1, _EMB)),
            wspec((_KW, _NS)), wspec((1, _NS)), wspec((1, _NS)),
        ],
        out_specs=(pl.BlockSpec((1, _HW, _EMB), lambda n: (n, 0, 0)),
                   pl.BlockSpec((1, _HW, _NS), lambda n: (n, 0, 0))),
        scratch_shapes=[
            vm(_CSP), vm(_CSP), vm(_CSP),
            vm(_CP), vm(_CP), vm(_CP),
            vm(_CP), vm(_CP), vm(_CP),
            vm(_CSP), vm(_CSP), vm(_CSP),
            pltpu.VMEM((_HW, _KW), bf),
        ],
        compiler_params=pltpu.CompilerParams(
            dimension_semantics=("parallel",)),
    )(cols, ws_n, ss_n, bs_n,
      w1_n, s1, b1, w2p.astype(bf), s2p, b2p,
      w_e, s_e, b_e, w_h, s_h, b_h)

    oh = oh.reshape(N, _H, _W, _NS)
    heatmap = oh[..., 0:_HM]
    offset = oh[..., _HM:_HM + 2]
    wh = oh[..., _HM + 2:_HM + 4]
    emb = oe.reshape(N, _H, _W, _EMB)
    to_nchw = lambda t: jnp.transpose(t, (0, 3, 1, 2))
    return to_nchw(heatmap), to_nchw(offset), to_nchw(wh), to_nchw(emb)
